# baseline (speedup 1.0000x reference)
import numpy as np

import jax
import jax.numpy as jnp
from jax.experimental import pallas as pl

_N = 65536
_C = 1000
_NBINS = 15

_delta = np.float32(_N) / np.float32(_NBINS)
_xq = np.arange(_NBINS + 1, dtype=np.float32) * _delta
_F = [int(np.floor(float(_xq[i]))) for i in range(1, _NBINS)]
_FRAC = [float(np.float32(float(_xq[i]) - np.floor(float(_xq[i]))))
         for i in range(1, _NBINS)]
_RANKS = [0] + [r for f in _F for r in (f, f + 1)] + [_N - 1]
_NR = len(_RANKS)


def _conf_acc_body(logits_ref, labels_ref, conf_ref, acc_ref):
    x = logits_ref[...]
    m = jnp.max(x, axis=1, keepdims=True)
    s = jnp.sum(jnp.exp(x - m), axis=1, keepdims=True)
    conf_ref[...] = 1.0 / s
    colids = jax.lax.broadcasted_iota(jnp.int32, x.shape, 1)
    ismax = x == m
    pred = jnp.min(jnp.where(ismax, colids, jnp.int32(_C)), axis=1,
                   keepdims=True)
    acc_ref[...] = (pred == labels_ref[...]).astype(jnp.float32)


def _ece_body(conf_ref, acc_ref, out_ref):
    conf = conf_ref[...]
    acc = acc_ref[...]
    bits = jax.lax.bitcast_convert_type(conf, jnp.int32)

    lo = [jnp.int32(0)] * _NR
    hi = [jnp.int32((1 << 30) - 1)] * _NR
    for _ in range(30):
        for j in range(_NR):
            mid = (lo[j] + hi[j]) >> 1
            cnt = jnp.sum((bits <= mid).astype(jnp.int32))
            take = cnt >= jnp.int32(_RANKS[j] + 1)
            hi[j] = jnp.where(take, mid, hi[j])
            lo[j] = jnp.where(take, lo[j], mid + jnp.int32(1))
    vals = jax.lax.bitcast_convert_type(jnp.stack(lo), jnp.float32)

    b = [None] * (_NBINS + 1)
    b[0] = vals[0]
    for i in range(1, _NBINS):
        vlo = vals[2 * i - 1]
        vhi = vals[2 * i]
        b[i] = vlo + jnp.float32(_FRAC[i - 1]) * (vhi - vlo)
    b[_NBINS] = vals[_NR - 1]

    ece = jnp.float32(0.0)
    mprev = (conf <= b[0]).astype(jnp.float32)
    cp = jnp.sum(mprev)
    sp = jnp.sum(conf * mprev)
    ap = jnp.sum(acc * mprev)
    for i in range(1, _NBINS + 1):
        mcur = (conf <= b[i]).astype(jnp.float32)
        cc = jnp.sum(mcur)
        sc = jnp.sum(conf * mcur)
        ac = jnp.sum(acc * mcur)
        cnt = cc - cp
        safe = jnp.maximum(cnt, 1.0)
        contrib = jnp.abs((sc - sp) / safe - (ac - ap) / safe) * (cnt / _N)
        ece = ece + jnp.where(cnt > 0, contrib, 0.0)
        cp, sp, ap = cc, sc, ac
    out_ref[...] = jnp.broadcast_to(ece, (1, 1))


def kernel(logits, labels):
    n, c = logits.shape
    rows = 1024
    grid = n // rows
    conf2d, acc2d = pl.pallas_call(
        _conf_acc_body,
        grid=(grid,),
        in_specs=[
            pl.BlockSpec((rows, c), lambda i: (i, 0)),
            pl.BlockSpec((rows, 1), lambda i: (i, 0)),
        ],
        out_specs=[
            pl.BlockSpec((rows, 1), lambda i: (i, 0)),
            pl.BlockSpec((rows, 1), lambda i: (i, 0)),
        ],
        out_shape=[
            jax.ShapeDtypeStruct((n, 1), jnp.float32),
            jax.ShapeDtypeStruct((n, 1), jnp.float32),
        ],
    )(logits, labels.reshape(n, 1))

    conf = conf2d.reshape(n // 128, 128)
    accv = acc2d.reshape(n // 128, 128)
    out = pl.pallas_call(
        _ece_body,
        in_specs=[
            pl.BlockSpec((n // 128, 128), lambda: (0, 0)),
            pl.BlockSpec((n // 128, 128), lambda: (0, 0)),
        ],
        out_specs=pl.BlockSpec((1, 1), lambda: (0, 0)),
        out_shape=jax.ShapeDtypeStruct((1, 1), jnp.float32),
    )(conf, accv)
    return out.reshape((1,))
